# baseline (speedup 1.0000x reference)
import functools

import jax
import jax.numpy as jnp
from jax import lax
from jax.experimental import pallas as pl
from jax.experimental.pallas import tpu as pltpu
from jax.experimental.pallas import tpu_sc as plsc

NSEG = 128
NC = 2
NS = 16
NW = NC * NS
SEGW = NSEG // NW
CHUNK = 192
STPAD = 160
DEAD = 159
SCANW = 3136


def _pool_sc(h, bpad):
    n, hid = h.shape
    fch = hid // 16
    half_f = fch // 2

    mesh = plsc.VectorSubcoreMesh(
        core_axis_name="c", subcore_axis_name="s",
        num_cores=NC, num_subcores=NS)

    @functools.partial(
        pl.kernel,
        out_type=jax.ShapeDtypeStruct((NW, SEGW, 3 * hid), jnp.float32),
        mesh=mesh,
        scratch_types=[
            pltpu.VMEM((CHUNK, hid), jnp.float32),
            pltpu.VMEM((CHUNK, hid), jnp.float32),
            pltpu.VMEM((STPAD,), jnp.int32),
            pltpu.VMEM((SEGW * hid,), jnp.float32),
            pltpu.VMEM((SEGW * hid,), jnp.float32),
            pltpu.VMEM((SEGW * hid,), jnp.float32),
            pltpu.VMEM((SEGW, 3 * hid), jnp.float32),
            pltpu.VMEM((16 + SCANW,), jnp.int32),
            pltpu.VMEM((16,), jnp.int32),
            pltpu.VMEM_SHARED((STPAD,), jnp.int32),
            pltpu.SemaphoreType.DMA,
            pltpu.SemaphoreType.DMA,
        ],
    )
    def k(h_hbm, bp_hbm, g3_hbm, buf0, buf1, st_v,
          acc_s, acc_q, acc_m, outb, bscan, valbuf, stsh, sem0, sem1):
        sid = lax.axis_index("s")
        wid = lax.axis_index("c") * NS + sid
        s0 = wid * SEGW
        iota16 = lax.iota(jnp.int32, 16)

        zeros_i = jnp.zeros((16,), jnp.int32)

        @pl.when(sid == 0)
        def _():
            for c in range(STPAD // 16):
                bscan[pl.ds(c * 16, 16)] = zeros_i
            pltpu.sync_copy(bscan.at[pl.ds(0, STPAD)], stsh)

        lo_t = ((sid * n // NS) // 8) * 8
        hi_t = (((sid + 1) * n // NS) // 8) * 8
        src_base = jnp.minimum(lo_t - 8, n - (8 + SCANW))

        @pl.when(lo_t == 0)
        def _():
            bscan[pl.ds(0, 16)] = jnp.full((16,), -1, jnp.int32)
            pltpu.sync_copy(bp_hbm.at[pl.ds(0, 8 + SCANW)],
                            bscan.at[pl.ds(8, 8 + SCANW)])

        @pl.when(lo_t > 0)
        def _():
            pltpu.sync_copy(bp_hbm.at[pl.ds(src_base, 8 + SCANW)],
                            bscan.at[pl.ds(0, 8 + SCANW)])

        idx0 = jnp.where(lo_t == 0, 8, lo_t - src_base)
        plsc.subcore_barrier()

        def scan_body(g, carry):
            o = idx0 + g * 16
            ids = bscan[pl.ds(o, 16)]
            prev = bscan[pl.ds(o - 1, 16)]
            rowv = lo_t + g * 16 + iota16
            isb = (ids != prev) & (rowv < hi_t)

            @pl.when(prev[0] != ids[15])
            def _():
                valbuf[pl.ds(0, 16)] = jnp.where(isb, rowv + 1, 0)
                idxv = jnp.where(isb, ids, DEAD)
                pltpu.sync_copy(valbuf, stsh.at[idxv], add=True)
            return carry

        lax.fori_loop(0, SCANW // 16, scan_body, 0)
        plsc.subcore_barrier()
        pltpu.sync_copy(stsh, st_v)

        big = jnp.float32(9.0e7)
        carry = jnp.full((16,), jnp.float32(n + 1))
        for c in range(8, -1, -1):
            raw = st_v[pl.ds(c * 16, 16)]
            enc = jnp.where(raw == 0, big, raw.astype(jnp.float32))
            v = enc
            for sh in (1, 2, 4, 8):
                idxs = jnp.minimum(iota16 + sh, 15)
                v = jnp.minimum(v, v.at[idxs].get(mode="promise_in_bounds"))
            v = jnp.minimum(v, carry)
            st_v[pl.ds(c * 16, 16)] = v.astype(jnp.int32) - 1
            carry = jnp.full((16,), v[0])

        zeros = jnp.zeros((16,), jnp.float32)
        ninf = jnp.full((16,), -jnp.inf, jnp.float32)
        for j in range(SEGW):
            for f in range(fch):
                acc_s[pl.ds(j * hid + f * 16, 16)] = zeros
                acc_q[pl.ds(j * hid + f * 16, 16)] = zeros
                acc_m[pl.ds(j * hid + f * 16, 16)] = ninf

        def sload(idx):
            return st_v[pl.ds(idx, 16)][0]

        bufs = ((buf0, sem0), (buf1, sem1))

        aj = [sload(s0 + j) for j in range(SEGW + 1)]
        a8j = [(aj[j] // 8) * 8 for j in range(SEGW)]
        nchj = [(aj[j + 1] - a8j[j] + (CHUNK - 1)) // CHUNK
                for j in range(SEGW)]
        cums = [jnp.int32(0)]
        for j in range(SEGW):
            cums.append(cums[-1] + nchj[j])
        tot = cums[SEGW]

        def selby(j, vals):
            r = vals[SEGW - 1]
            for t in range(SEGW - 2, -1, -1):
                r = jnp.where(j == t, vals[t], r)
            return r

        def sched(k):
            j = ((k >= cums[1]).astype(jnp.int32)
                 + (k >= cums[2]).astype(jnp.int32)
                 + (k >= cums[3]).astype(jnp.int32))
            kk = k - selby(j, cums[:SEGW])
            r0 = selby(j, a8j) + kk * CHUNK
            base = jnp.minimum(r0, n - CHUNK)
            lo = jnp.maximum(r0, selby(j, aj[:SEGW])) - base
            hi = jnp.minimum(selby(j, aj[1:]) - base, CHUNK)
            return j, base, lo, hi

        def start_dma(base, bf, sm):
            pltpu.async_copy(h_hbm.at[pl.ds(base, CHUNK)], bf, sm)

        def wait_dma(base, bf, sm):
            pltpu.make_async_copy(
                h_hbm.at[pl.ds(base, CHUNK)], bf, sm).wait()

        def process(jdyn, bf, lo, hi):
            init = (tuple(zeros for _ in range(fch)),
                    tuple(zeros for _ in range(fch)),
                    tuple(ninf for _ in range(fch)))

            @plsc.parallel_loop(lo, hi, carry=init, unroll=2)
            def acc_loop(i, car, bf=bf):
                ns_, nq_, nm_ = [], [], []
                for f in range(fch):
                    v = bf[i, pl.ds(f * 16, 16)]
                    ns_.append(car[0][f] + v)
                    nq_.append(car[1][f] + v * v)
                    nm_.append(jnp.maximum(car[2][f], v))
                return (tuple(ns_), tuple(nq_), tuple(nm_))

            ss, qq, mm = acc_loop
            ob = jdyn * hid
            for f in range(fch):
                o = ob + f * 16
                acc_s[pl.ds(o, 16)] = acc_s[pl.ds(o, 16)] + ss[f]
                acc_q[pl.ds(o, 16)] = acc_q[pl.ds(o, 16)] + qq[f]
                acc_m[pl.ds(o, 16)] = jnp.maximum(
                    acc_m[pl.ds(o, 16)], mm[f])

        @pl.when(tot > 0)
        def _():
            _, b0, _, _ = sched(jnp.int32(0))
            start_dma(b0, buf0, sem0)

        def pair_body(g, carry):
            for b in range(2):
                bf, sm = bufs[b]
                obf, osm = bufs[1 - b]
                k = 2 * g + b

                @pl.when(k < tot)
                def _(k=k, bf=bf, sm=sm, obf=obf, osm=osm):
                    j, base, lo, hi = sched(k)
                    wait_dma(base, bf, sm)

                    @pl.when(k + 1 < tot)
                    def _():
                        _, nb, _, _ = sched(k + 1)
                        start_dma(nb, obf, osm)

                    process(j, bf, lo, hi)
            return carry

        lax.fori_loop(0, (tot + 1) // 2, pair_body, 0)

        for j in range(SEGW):
            cntf = (aj[j + 1] - aj[j]).astype(jnp.float32)
            inv = jnp.ones((16,), jnp.float32) / jnp.full(
                (16,), jnp.maximum(cntf, 1.0), jnp.float32)
            for f in range(fch):
                o = j * hid + f * 16
                s_ = acc_s[pl.ds(o, 16)]
                q_ = acc_q[pl.ds(o, 16)]
                m_ = acc_m[pl.ds(o, 16)]
                mean = s_ * inv
                var = jnp.maximum(q_ * inv - mean * mean, 0.0)
                outb[j, pl.ds(f * 16, 16)] = mean
                outb[j, pl.ds(hid + f * 16, 16)] = m_
                outb[j, pl.ds(2 * hid + f * 16, 16)] = var

        pltpu.sync_copy(outb, g3_hbm.at[wid])

    return k(h, bpad)


def _mlp_tc(g4, W1, b1, W2, b2):
    nseg = g4.shape[0] * g4.shape[1]
    hid = g4.shape[2] // 3
    nq = W2.shape[1]

    def body(g3_ref, w1_ref, b1_ref, w2_ref, b2_ref, z_ref):
        g3v = g3_ref[...].reshape(nseg, 3 * hid)
        std = jnp.sqrt(g3v[:, 2 * hid:] + 1e-8)
        g = jnp.concatenate([g3v[:, :2 * hid], std], axis=1)
        hdn = jnp.maximum(
            jnp.dot(g, w1_ref[...], preferred_element_type=jnp.float32)
            + b1_ref[...], 0.0)
        z = jnp.tanh(
            jnp.dot(hdn, w2_ref[...], preferred_element_type=jnp.float32)
            + b2_ref[...])
        z_ref[...] = z * jnp.float32(jnp.pi)

    return pl.pallas_call(
        body,
        out_shape=jax.ShapeDtypeStruct((nseg, nq), jnp.float32),
    )(g4, W1, b1.reshape(1, -1), W2, b2.reshape(1, -1))


def kernel(h, W1, b1, W2, b2, batch):
    b32 = batch.astype(jnp.int32)
    g4 = _pool_sc(h, b32)
    return _mlp_tc(g4, W1, b1, W2, b2)


# speedup vs baseline: 1.0035x; 1.0035x over previous
import functools

import jax
import jax.numpy as jnp
from jax import lax
from jax.experimental import pallas as pl
from jax.experimental.pallas import tpu as pltpu
from jax.experimental.pallas import tpu_sc as plsc

NSEG = 128
NC = 2
NS = 16
NW = NC * NS
SEGW = NSEG // NW
CHUNK = 192
STPAD = 160
DEAD = 159
SCANW = 3136


def _pool_sc(h, bpad):
    n, hid = h.shape
    fch = hid // 16
    half_f = fch // 2

    mesh = plsc.VectorSubcoreMesh(
        core_axis_name="c", subcore_axis_name="s",
        num_cores=NC, num_subcores=NS)

    @functools.partial(
        pl.kernel,
        out_type=jax.ShapeDtypeStruct((NW, SEGW, 3 * hid), jnp.float32),
        mesh=mesh,
        scratch_types=[
            pltpu.VMEM((CHUNK, hid), jnp.float32),
            pltpu.VMEM((CHUNK, hid), jnp.float32),
            pltpu.VMEM((STPAD,), jnp.int32),
            pltpu.VMEM((SEGW * hid,), jnp.float32),
            pltpu.VMEM((SEGW * hid,), jnp.float32),
            pltpu.VMEM((SEGW * hid,), jnp.float32),
            pltpu.VMEM((SEGW, 3 * hid), jnp.float32),
            pltpu.VMEM((16 + SCANW,), jnp.int32),
            pltpu.VMEM((16,), jnp.int32),
            pltpu.VMEM_SHARED((STPAD,), jnp.int32),
            pltpu.SemaphoreType.DMA,
            pltpu.SemaphoreType.DMA,
        ],
    )
    def k(h_hbm, bp_hbm, g3_hbm, buf0, buf1, st_v,
          acc_s, acc_q, acc_m, outb, bscan, valbuf, stsh, sem0, sem1):
        sid = lax.axis_index("s")
        wid = lax.axis_index("c") * NS + sid
        s0 = wid * SEGW
        iota16 = lax.iota(jnp.int32, 16)

        zeros_i = jnp.zeros((16,), jnp.int32)

        @pl.when(sid == 0)
        def _():
            for c in range(STPAD // 16):
                bscan[pl.ds(c * 16, 16)] = zeros_i
            pltpu.sync_copy(bscan.at[pl.ds(0, STPAD)], stsh)

        lo_t = ((sid * n // NS) // 8) * 8
        hi_t = (((sid + 1) * n // NS) // 8) * 8
        src_base = jnp.minimum(lo_t - 8, n - (8 + SCANW))
        bscan[pl.ds(SCANW, 16)] = jnp.full((16,), 1 << 20, jnp.int32)

        @pl.when(lo_t == 0)
        def _():
            bscan[pl.ds(0, 16)] = jnp.full((16,), -1, jnp.int32)
            pltpu.sync_copy(bp_hbm.at[pl.ds(0, 8 + SCANW)],
                            bscan.at[pl.ds(8, 8 + SCANW)])

        @pl.when(lo_t > 0)
        def _():
            pltpu.sync_copy(bp_hbm.at[pl.ds(src_base, 8 + SCANW)],
                            bscan.at[pl.ds(0, 8 + SCANW)])

        idx0 = jnp.where(lo_t == 0, 8, lo_t - src_base)
        plsc.subcore_barrier()

        def scan_body(g, carry):
            o = idx0 + g * 16
            ids = bscan[pl.ds(o, 16)]
            prev = bscan[pl.ds(o - 1, 16)]
            rowv = lo_t + g * 16 + iota16
            isb = (ids != prev) & (rowv < hi_t)

            @pl.when(prev[0] != ids[15])
            def _():
                valbuf[pl.ds(0, 16)] = jnp.where(isb, rowv + 1, 0)
                idxv = jnp.where(isb, ids, DEAD)
                pltpu.sync_copy(valbuf, stsh.at[idxv], add=True)
            return carry

        lax.fori_loop(0, SCANW // 16, scan_body, 0)
        plsc.subcore_barrier()
        pltpu.sync_copy(stsh, st_v)

        big = jnp.float32(9.0e7)
        carry = jnp.full((16,), jnp.float32(n + 1))
        for c in range(8, -1, -1):
            raw = st_v[pl.ds(c * 16, 16)]
            enc = jnp.where(raw == 0, big, raw.astype(jnp.float32))
            v = enc
            for sh in (1, 2, 4, 8):
                idxs = jnp.minimum(iota16 + sh, 15)
                v = jnp.minimum(v, v.at[idxs].get(mode="promise_in_bounds"))
            v = jnp.minimum(v, carry)
            st_v[pl.ds(c * 16, 16)] = v.astype(jnp.int32) - 1
            carry = jnp.full((16,), v[0])

        zeros = jnp.zeros((16,), jnp.float32)
        ninf = jnp.full((16,), -jnp.inf, jnp.float32)
        for j in range(SEGW):
            for f in range(fch):
                acc_s[pl.ds(j * hid + f * 16, 16)] = zeros
                acc_q[pl.ds(j * hid + f * 16, 16)] = zeros
                acc_m[pl.ds(j * hid + f * 16, 16)] = ninf

        def sload(idx):
            return st_v[pl.ds(idx, 16)][0]

        bufs = ((buf0, sem0), (buf1, sem1))

        aj = [sload(s0 + j) for j in range(SEGW + 1)]
        a8j = [(aj[j] // 8) * 8 for j in range(SEGW)]
        nchj = [(aj[j + 1] - a8j[j] + (CHUNK - 1)) // CHUNK
                for j in range(SEGW)]
        cums = [jnp.int32(0)]
        for j in range(SEGW):
            cums.append(cums[-1] + nchj[j])
        tot = cums[SEGW]

        def selby(j, vals):
            r = vals[SEGW - 1]
            for t in range(SEGW - 2, -1, -1):
                r = jnp.where(j == t, vals[t], r)
            return r

        def sched(k):
            j = ((k >= cums[1]).astype(jnp.int32)
                 + (k >= cums[2]).astype(jnp.int32)
                 + (k >= cums[3]).astype(jnp.int32))
            kk = k - selby(j, cums[:SEGW])
            r0 = selby(j, a8j) + kk * CHUNK
            base = jnp.minimum(r0, n - CHUNK)
            lo = jnp.maximum(r0, selby(j, aj[:SEGW])) - base
            hi = jnp.minimum(selby(j, aj[1:]) - base, CHUNK)
            return j, base, lo, hi

        def start_dma(base, bf, sm):
            pltpu.async_copy(h_hbm.at[pl.ds(base, CHUNK)], bf, sm)

        def wait_dma(base, bf, sm):
            pltpu.make_async_copy(
                h_hbm.at[pl.ds(base, CHUNK)], bf, sm).wait()

        def process(jdyn, bf, lo, hi):
            init = (tuple(zeros for _ in range(fch)),
                    tuple(zeros for _ in range(fch)),
                    tuple(ninf for _ in range(fch)))

            @plsc.parallel_loop(lo, hi, carry=init, unroll=2)
            def acc_loop(i, car, bf=bf):
                ns_, nq_, nm_ = [], [], []
                for f in range(fch):
                    v = bf[i, pl.ds(f * 16, 16)]
                    ns_.append(car[0][f] + v)
                    nq_.append(car[1][f] + v * v)
                    nm_.append(jnp.maximum(car[2][f], v))
                return (tuple(ns_), tuple(nq_), tuple(nm_))

            ss, qq, mm = acc_loop
            ob = jdyn * hid
            for f in range(fch):
                o = ob + f * 16
                acc_s[pl.ds(o, 16)] = acc_s[pl.ds(o, 16)] + ss[f]
                acc_q[pl.ds(o, 16)] = acc_q[pl.ds(o, 16)] + qq[f]
                acc_m[pl.ds(o, 16)] = jnp.maximum(
                    acc_m[pl.ds(o, 16)], mm[f])

        @pl.when(tot > 0)
        def _():
            _, b0, _, _ = sched(jnp.int32(0))
            start_dma(b0, buf0, sem0)

        def pair_body(g, carry):
            for b in range(2):
                bf, sm = bufs[b]
                obf, osm = bufs[1 - b]
                k = 2 * g + b

                @pl.when(k < tot)
                def _(k=k, bf=bf, sm=sm, obf=obf, osm=osm):
                    j, base, lo, hi = sched(k)
                    wait_dma(base, bf, sm)

                    @pl.when(k + 1 < tot)
                    def _():
                        _, nb, _, _ = sched(k + 1)
                        start_dma(nb, obf, osm)

                    process(j, bf, lo, hi)
            return carry

        lax.fori_loop(0, (tot + 1) // 2, pair_body, 0)

        for j in range(SEGW):
            cntf = (aj[j + 1] - aj[j]).astype(jnp.float32)
            inv = jnp.ones((16,), jnp.float32) / jnp.full(
                (16,), jnp.maximum(cntf, 1.0), jnp.float32)
            for f in range(fch):
                o = j * hid + f * 16
                s_ = acc_s[pl.ds(o, 16)]
                q_ = acc_q[pl.ds(o, 16)]
                m_ = acc_m[pl.ds(o, 16)]
                mean = s_ * inv
                var = jnp.maximum(q_ * inv - mean * mean, 0.0)
                outb[j, pl.ds(f * 16, 16)] = mean
                outb[j, pl.ds(hid + f * 16, 16)] = m_
                outb[j, pl.ds(2 * hid + f * 16, 16)] = var

        pltpu.sync_copy(outb, g3_hbm.at[wid])

    return k(h, bpad)


def _mlp_tc(g4, W1, b1, W2, b2):
    nseg = g4.shape[0] * g4.shape[1]
    hid = g4.shape[2] // 3
    nq = W2.shape[1]

    def body(g3_ref, w1_ref, b1_ref, w2_ref, b2_ref, z_ref):
        g3v = g3_ref[...].reshape(nseg, 3 * hid)
        std = jnp.sqrt(g3v[:, 2 * hid:] + 1e-8)
        g = jnp.concatenate([g3v[:, :2 * hid], std], axis=1)
        hdn = jnp.maximum(
            jnp.dot(g, w1_ref[...], preferred_element_type=jnp.float32)
            + b1_ref[...], 0.0)
        z = jnp.tanh(
            jnp.dot(hdn, w2_ref[...], preferred_element_type=jnp.float32)
            + b2_ref[...])
        z_ref[...] = z * jnp.float32(jnp.pi)

    return pl.pallas_call(
        body,
        out_shape=jax.ShapeDtypeStruct((nseg, nq), jnp.float32),
    )(g4, W1, b1.reshape(1, -1), W2, b2.reshape(1, -1))


def kernel(h, W1, b1, W2, b2, batch):
    b32 = batch.astype(jnp.int32)
    g4 = _pool_sc(h, b32)
    return _mlp_tc(g4, W1, b1, W2, b2)
